# baseline (speedup 1.0000x reference)
import functools

import jax
import jax.numpy as jnp
from jax import lax
from jax.experimental import pallas as pl
from jax.experimental.pallas import tpu as pltpu
from jax.experimental.pallas import tpu_sc as plsc

_LANES = 16
_NBUF = 12


@functools.lru_cache(maxsize=None)
def _build(B, K, N, NC, NS):
    NW = NC * NS
    b_per_w = B // NW
    n_groups = b_per_w // _LANES
    kb_n = K // 8
    mesh = plsc.VectorSubcoreMesh(core_axis_name="c", subcore_axis_name="s")

    @functools.partial(
        pl.kernel,
        mesh=mesh,
        compiler_params=pltpu.CompilerParams(needs_layout_passes=False),
        out_type=jax.ShapeDtypeStruct((B,), jnp.float32),
        scratch_types=[
            pltpu.VMEM((b_per_w,), jnp.int32),
            pltpu.VMEM((b_per_w,), jnp.int32),
            pltpu.VMEM((_NBUF, K // 8, 8, 128), jnp.float32),
            pltpu.VMEM((_NBUF, K // 8, 8, 128), jnp.float32),
            pltpu.VMEM((b_per_w,), jnp.float32),
            pltpu.SemaphoreType.DMA,
            pltpu.SemaphoreType.DMA,
        ],
    )
    def sc_kernel(u_hbm, i_hbm, pt_hbm, qt_hbm, out_hbm,
                  u_v, i_v, ps_v, qs_v, out_v, sem_p, sem_q):
        wid = lax.axis_index("s") * NC + lax.axis_index("c")
        base = wid * b_per_w

        pltpu.sync_copy(u_hbm.at[pl.ds(base, b_per_w)], u_v)
        pltpu.sync_copy(i_hbm.at[pl.ds(base, b_per_w)], i_v)

        lanes = lax.broadcasted_iota(jnp.int32, (_LANES,), 0)
        lane_last = jnp.full((_LANES,), _LANES - 1, jnp.int32)

        def enqueue(g, r, buf):
            u16 = u_v[pl.ds(g * _LANES, _LANES)]
            i16 = i_v[pl.ds(g * _LANES, _LANES)]
            ur = u16[r]
            ir = i16[r]
            ub = pl.multiple_of((ur >> 7) << 7, 128)
            ib = pl.multiple_of((ir >> 7) << 7, 128)
            pltpu.async_copy(
                pt_hbm.at[:, :, pl.ds(ub, 128)],
                ps_v.at[buf], sem_p)
            pltpu.async_copy(
                qt_hbm.at[:, :, pl.ds(ib, 128)],
                qs_v.at[buf], sem_q)

        def drain(buf):
            pltpu.make_async_copy(
                pt_hbm.at[:, :, pl.ds(0, 128)],
                ps_v.at[buf], sem_p).wait()
            pltpu.make_async_copy(
                qt_hbm.at[:, :, pl.ds(0, 128)],
                qs_v.at[buf], sem_q).wait()

        def compute(g, r, buf, acc16):
            u16 = u_v[pl.ds(g * _LANES, _LANES)]
            i16 = i_v[pl.ds(g * _LANES, _LANES)]
            ul = jnp.full((_LANES,), u16[r] & 127, jnp.int32)
            il = jnp.full((_LANES,), i16[r] & 127, jnp.int32)
            bufv = jnp.full((_LANES,), buf, jnp.int32)
            acc = jnp.zeros((_LANES,), jnp.float32)
            for h in range(K // _LANES):
                ks = h * _LANES + lanes
                a = plsc.load_gather(ps_v, [bufv, ks >> 3, ks & 7, ul])
                b = plsc.load_gather(qs_v, [bufv, ks >> 3, ks & 7, il])
                acc = acc + a * b
            csum = plsc.cumsum(acc)
            tot = csum.at[lane_last].get(mode="promise_in_bounds")
            return jnp.where(lanes == r, tot, acc16)

        for r in range(_NBUF):
            enqueue(0, r, r)

        def group_body(g, carry):
            acc16 = jnp.zeros((_LANES,), jnp.float32)
            for r in range(_LANES):
                buf = r % _NBUF
                drain(buf)
                acc16 = compute(g, r, buf, acc16)
                nr = (r + _NBUF) % _LANES
                ng = g if r + _NBUF < _LANES else g + 1

                @pl.when(ng < n_groups)
                def _():
                    enqueue(ng, nr, buf)

            out_v[pl.ds(g * _LANES, _LANES)] = acc16
            return carry

        lax.fori_loop(0, n_groups, group_body, 0)

        pltpu.sync_copy(out_v, out_hbm.at[pl.ds(base, b_per_w)])

    return sc_kernel


def kernel(u, i, P, Q, mu):
    B = u.shape[0]
    N, K = P.shape
    info = plsc.get_sparse_core_info()
    sc = _build(B, K, N, info.num_cores, info.num_subcores)
    pt = P.T.reshape(K // 8, 8, N)
    qt = Q.T.reshape(K // 8, 8, N)
    out = sc(u.astype(jnp.int32), i.astype(jnp.int32), pt, qt)
    return out + mu


# speedup vs baseline: 1.0199x; 1.0199x over previous
import functools

import jax
import jax.numpy as jnp
from jax import lax
from jax.experimental import pallas as pl
from jax.experimental.pallas import tpu as pltpu
from jax.experimental.pallas import tpu_sc as plsc

_LANES = 16
_NBUF = 8


@functools.lru_cache(maxsize=None)
def _build(B, K, N, NC, NS):
    NW = NC * NS
    b_per_w = B // NW
    n_groups = b_per_w // _LANES
    kb_n = K // 8
    mesh = plsc.VectorSubcoreMesh(core_axis_name="c", subcore_axis_name="s")

    @functools.partial(
        pl.kernel,
        mesh=mesh,
        compiler_params=pltpu.CompilerParams(needs_layout_passes=False),
        out_type=jax.ShapeDtypeStruct((B,), jnp.float32),
        scratch_types=[
            pltpu.VMEM((b_per_w,), jnp.int32),
            pltpu.VMEM((b_per_w,), jnp.int32),
            pltpu.VMEM((_NBUF, K // 8, 8, 128), jnp.float32),
            pltpu.VMEM((_NBUF, K // 8, 8, 128), jnp.float32),
            pltpu.VMEM((b_per_w,), jnp.float32),
            pltpu.VMEM((16,), jnp.float32),
            pltpu.SemaphoreType.DMA,
            pltpu.SemaphoreType.DMA,
        ],
    )
    def sc_kernel(u_hbm, i_hbm, pt_hbm, qt_hbm, mu_hbm, out_hbm,
                  u_v, i_v, ps_v, qs_v, out_v, mu_v, sem_p, sem_q):
        wid = lax.axis_index("s") * NC + lax.axis_index("c")
        base = wid * b_per_w

        pltpu.sync_copy(u_hbm.at[pl.ds(base, b_per_w)], u_v)
        pltpu.sync_copy(i_hbm.at[pl.ds(base, b_per_w)], i_v)
        pltpu.sync_copy(mu_hbm, mu_v.at[pl.ds(0, 1)])

        lanes = lax.broadcasted_iota(jnp.int32, (_LANES,), 0)
        lane_last = jnp.full((_LANES,), _LANES - 1, jnp.int32)
        mu16 = plsc.load_gather(mu_v, [jnp.zeros((_LANES,), jnp.int32)])

        def enqueue(g, r, buf):
            u16 = u_v[pl.ds(g * _LANES, _LANES)]
            i16 = i_v[pl.ds(g * _LANES, _LANES)]
            ur = u16[r]
            ir = i16[r]
            ub = pl.multiple_of((ur >> 7) << 7, 128)
            ib = pl.multiple_of((ir >> 7) << 7, 128)
            pltpu.async_copy(
                pt_hbm.at[:, :, pl.ds(ub, 128)],
                ps_v.at[buf], sem_p)
            pltpu.async_copy(
                qt_hbm.at[:, :, pl.ds(ib, 128)],
                qs_v.at[buf], sem_q)

        def drain(buf):
            pltpu.make_async_copy(
                pt_hbm.at[:, :, pl.ds(0, 128)],
                ps_v.at[buf], sem_p).wait()
            pltpu.make_async_copy(
                qt_hbm.at[:, :, pl.ds(0, 128)],
                qs_v.at[buf], sem_q).wait()

        def compute(g, r, buf, acc16):
            u16 = u_v[pl.ds(g * _LANES, _LANES)]
            i16 = i_v[pl.ds(g * _LANES, _LANES)]
            ul = jnp.full((_LANES,), u16[r] & 127, jnp.int32)
            il = jnp.full((_LANES,), i16[r] & 127, jnp.int32)
            bufv = jnp.full((_LANES,), buf, jnp.int32)
            acc = jnp.zeros((_LANES,), jnp.float32)
            for h in range(K // _LANES):
                ks = h * _LANES + lanes
                a = plsc.load_gather(ps_v, [bufv, ks >> 3, ks & 7, ul])
                b = plsc.load_gather(qs_v, [bufv, ks >> 3, ks & 7, il])
                acc = acc + a * b
            csum = plsc.cumsum(acc)
            tot = csum.at[lane_last].get(mode="promise_in_bounds")
            return jnp.where(lanes == r, tot, acc16)

        for r in range(_NBUF):
            enqueue(0, r, r)

        def group_body(g, carry):
            acc16 = jnp.zeros((_LANES,), jnp.float32)
            for r in range(_LANES):
                buf = r % _NBUF
                drain(buf)
                acc16 = compute(g, r, buf, acc16)
                nr = (r + _NBUF) % _LANES
                ng = g if r + _NBUF < _LANES else g + 1

                @pl.when(ng < n_groups)
                def _():
                    enqueue(ng, nr, buf)

            out_v[pl.ds(g * _LANES, _LANES)] = acc16 + mu16
            return carry

        lax.fori_loop(0, n_groups, group_body, 0)

        pltpu.sync_copy(out_v, out_hbm.at[pl.ds(base, b_per_w)])

    return sc_kernel


def kernel(u, i, P, Q, mu):
    B = u.shape[0]
    N, K = P.shape
    info = plsc.get_sparse_core_info()
    sc = _build(B, K, N, info.num_cores, info.num_subcores)
    pt = P.T.reshape(K // 8, 8, N)
    qt = Q.T.reshape(K // 8, 8, N)
    return sc(u.astype(jnp.int32), i.astype(jnp.int32), pt, qt,
              mu.astype(jnp.float32))
